# baseline (speedup 1.0000x reference)
import functools

import jax
import jax.numpy as jnp
from jax import lax
from jax.experimental import pallas as pl
from jax.experimental.pallas import tpu as pltpu
from jax.experimental.pallas import tpu_sc as plsc

_B = 4096
_L = 20
_D = 16
_NC = 2
_NS = 16
_NW = _NC * _NS
_BPW = _B // _NW

_V = 1000000
_CPB = 116736
_GRID = (_V + _CPB - 1) // _CPB
_ROWS = _GRID * (_CPB // 128)

_mesh = plsc.VectorSubcoreMesh(core_axis_name="c", subcore_axis_name="s")


def _tdot_body(tt_ref, wb_ref, out_ref):
    acc = tt_ref[...] * wb_ref[0:16, :]
    s = jnp.sum(acc, axis=0, keepdims=True)
    for r in range(_CPB // 128):
        out_ref[r:r + 1, :] = s[:, r * 128:(r + 1) * 128] + wb_ref[16:17, :]


def _tdot(table_t, wb):
    return pl.pallas_call(
        _tdot_body,
        grid=(_GRID,),
        in_specs=[
            pl.BlockSpec((_D, _CPB), lambda g: (0, g)),
            pl.BlockSpec((_D + 1, 1), lambda g: (0, 0)),
        ],
        out_specs=pl.BlockSpec((_CPB // 128, 128), lambda g: (g, 0)),
        out_shape=jax.ShapeDtypeStruct((_ROWS, 128), jnp.float32),
    )(table_t, wb)


@functools.partial(
    pl.kernel,
    mesh=_mesh,
    out_type=jax.ShapeDtypeStruct((_B,), jnp.float32),
    scratch_types=[
        pltpu.VMEM((_L * _BPW,), jnp.int32),
        pltpu.VMEM((_L * _BPW,), jnp.float32),
        pltpu.VMEM((_BPW,), jnp.float32),
        pltpu.SemaphoreType.DMA,
        pltpu.SemaphoreType.DMA,
    ],
    compiler_params=pltpu.CompilerParams(
        needs_layout_passes=False, use_tc_tiling_on_sc=False),
)
def _sc_pool(idx_hbm, tdot_hbm, out_hbm, idx_v, val_v, out_v, sem, sem_b):
    wid = lax.axis_index("s") * _NC + lax.axis_index("c")
    base = wid * _BPW

    half = _L // 2 * _BPW
    pltpu.sync_copy(idx_hbm.at[wid], idx_v)
    cp_a = pltpu.async_copy(
        tdot_hbm.at[idx_v.at[pl.ds(0, half)]], val_v.at[pl.ds(0, half)], sem)
    cp_b = pltpu.async_copy(
        tdot_hbm.at[idx_v.at[pl.ds(half, half)]],
        val_v.at[pl.ds(half, half)], sem_b)

    cp_a.wait()
    accs = []
    for c in range(_BPW // 16):
        acc = val_v[pl.ds(c * 16, 16)]
        for l in range(1, _L // 2):
            acc = acc + val_v[pl.ds(l * _BPW + c * 16, 16)]
        accs.append(acc)
    cp_b.wait()
    for c in range(_BPW // 16):
        acc = accs[c]
        for l in range(_L // 2, _L):
            acc = acc + val_v[pl.ds(l * _BPW + c * 16, 16)]
        out_v[pl.ds(c * 16, 16)] = 1.0 / (1.0 + jnp.exp(-acc))

    pltpu.sync_copy(out_v, out_hbm.at[pl.ds(base, _BPW)])


def kernel(x, table, W, b):
    table_t = table.astype(jnp.float32).T
    wb = jnp.concatenate(
        [W.astype(jnp.float32),
         b.astype(jnp.float32).reshape(1, 1) * (1.0 / _L)], axis=0)
    tdot = _tdot(table_t, wb)
    idx = (x.astype(jnp.int32)
           .reshape(_NW, _BPW, _L)
           .transpose(0, 2, 1)
           .reshape(_NW, _L * _BPW))
    out = _sc_pool(idx, tdot.reshape(-1))
    return out.reshape(_B, 1)


# speedup vs baseline: 1.0063x; 1.0063x over previous
import functools

import jax
import jax.numpy as jnp
from jax import lax
from jax.experimental import pallas as pl
from jax.experimental.pallas import tpu as pltpu
from jax.experimental.pallas import tpu_sc as plsc

_B = 4096
_L = 20
_D = 16
_NC = 2
_NS = 16
_NW = _NC * _NS
_BPW = _B // _NW

_V = 1000000
_CPB = 149504
_GRID = (_V + _CPB - 1) // _CPB
_ROWS = _GRID * (_CPB // 128)

_mesh = plsc.VectorSubcoreMesh(core_axis_name="c", subcore_axis_name="s")


def _tdot_body(tt_ref, wb_ref, out_ref):
    acc = tt_ref[...] * wb_ref[0:16, :]
    s = jnp.sum(acc, axis=0, keepdims=True)
    for r in range(_CPB // 128):
        out_ref[r:r + 1, :] = s[:, r * 128:(r + 1) * 128] + wb_ref[16:17, :]


def _tdot(table_t, wb):
    return pl.pallas_call(
        _tdot_body,
        grid=(_GRID,),
        in_specs=[
            pl.BlockSpec((_D, _CPB), lambda g: (0, g)),
            pl.BlockSpec((_D + 1, 1), lambda g: (0, 0)),
        ],
        out_specs=pl.BlockSpec((_CPB // 128, 128), lambda g: (g, 0)),
        out_shape=jax.ShapeDtypeStruct((_ROWS, 128), jnp.float32),
    )(table_t, wb)


@functools.partial(
    pl.kernel,
    mesh=_mesh,
    out_type=jax.ShapeDtypeStruct((_B,), jnp.float32),
    scratch_types=[
        pltpu.VMEM((_L * _BPW,), jnp.int32),
        pltpu.VMEM((_L * _BPW,), jnp.float32),
        pltpu.VMEM((_BPW,), jnp.float32),
        pltpu.SemaphoreType.DMA,
        pltpu.SemaphoreType.DMA,
    ],
    compiler_params=pltpu.CompilerParams(
        needs_layout_passes=False, use_tc_tiling_on_sc=False),
)
def _sc_pool(idx_hbm, tdot_hbm, out_hbm, idx_v, val_v, out_v, sem, sem_b):
    wid = lax.axis_index("s") * _NC + lax.axis_index("c")
    base = wid * _BPW

    half = _L // 2 * _BPW
    pltpu.sync_copy(idx_hbm.at[wid], idx_v)
    cp_a = pltpu.async_copy(
        tdot_hbm.at[idx_v.at[pl.ds(0, half)]], val_v.at[pl.ds(0, half)], sem)
    cp_b = pltpu.async_copy(
        tdot_hbm.at[idx_v.at[pl.ds(half, half)]],
        val_v.at[pl.ds(half, half)], sem_b)

    cp_a.wait()
    accs = []
    for c in range(_BPW // 16):
        acc = val_v[pl.ds(c * 16, 16)]
        for l in range(1, _L // 2):
            acc = acc + val_v[pl.ds(l * _BPW + c * 16, 16)]
        accs.append(acc)
    cp_b.wait()
    for c in range(_BPW // 16):
        acc = accs[c]
        for l in range(_L // 2, _L):
            acc = acc + val_v[pl.ds(l * _BPW + c * 16, 16)]
        out_v[pl.ds(c * 16, 16)] = 1.0 / (1.0 + jnp.exp(-acc))

    pltpu.sync_copy(out_v, out_hbm.at[pl.ds(base, _BPW)])


def kernel(x, table, W, b):
    table_t = table.astype(jnp.float32).T
    wb = jnp.concatenate(
        [W.astype(jnp.float32),
         b.astype(jnp.float32).reshape(1, 1) * (1.0 / _L)], axis=0)
    tdot = _tdot(table_t, wb)
    idx = (x.astype(jnp.int32)
           .reshape(_NW, _BPW, _L)
           .transpose(0, 2, 1)
           .reshape(_NW, _L * _BPW))
    out = _sc_pool(idx, tdot.reshape(-1))
    return out.reshape(_B, 1)


# speedup vs baseline: 1.0067x; 1.0004x over previous
import functools

import jax
import jax.numpy as jnp
from jax import lax
from jax.experimental import pallas as pl
from jax.experimental.pallas import tpu as pltpu
from jax.experimental.pallas import tpu_sc as plsc

_B = 4096
_L = 20
_D = 16
_NC = 2
_NS = 16
_NW = _NC * _NS
_BPW = _B // _NW

_V = 1000000
_CPB = 131072
_GRID = (_V + _CPB - 1) // _CPB
_ROWS = _GRID * (_CPB // 128)

_mesh = plsc.VectorSubcoreMesh(core_axis_name="c", subcore_axis_name="s")


def _tdot_body(tt_ref, wb_ref, out_ref):
    acc = tt_ref[...] * wb_ref[0:16, :]
    s = jnp.sum(acc, axis=0, keepdims=True)
    for r in range(_CPB // 128):
        out_ref[r:r + 1, :] = s[:, r * 128:(r + 1) * 128] + wb_ref[16:17, :]


def _tdot(table_t, wb):
    return pl.pallas_call(
        _tdot_body,
        grid=(_GRID,),
        in_specs=[
            pl.BlockSpec((_D, _CPB), lambda g: (0, g)),
            pl.BlockSpec((_D + 1, 1), lambda g: (0, 0)),
        ],
        out_specs=pl.BlockSpec((_CPB // 128, 128), lambda g: (g, 0)),
        out_shape=jax.ShapeDtypeStruct((_ROWS, 128), jnp.float32),
    )(table_t, wb)


@functools.partial(
    pl.kernel,
    mesh=_mesh,
    out_type=jax.ShapeDtypeStruct((_B,), jnp.float32),
    scratch_types=[
        pltpu.VMEM((_L * _BPW,), jnp.int32),
        pltpu.VMEM((_L * _BPW,), jnp.float32),
        pltpu.VMEM((_BPW,), jnp.float32),
        pltpu.SemaphoreType.DMA,
        pltpu.SemaphoreType.DMA,
    ],
    compiler_params=pltpu.CompilerParams(
        needs_layout_passes=False, use_tc_tiling_on_sc=False),
)
def _sc_pool(idx_hbm, tdot_hbm, out_hbm, idx_v, val_v, out_v, sem, sem_b):
    wid = lax.axis_index("s") * _NC + lax.axis_index("c")
    base = wid * _BPW

    half = _L // 2 * _BPW
    pltpu.sync_copy(idx_hbm.at[wid], idx_v)
    cp_a = pltpu.async_copy(
        tdot_hbm.at[idx_v.at[pl.ds(0, half)]], val_v.at[pl.ds(0, half)], sem)
    cp_b = pltpu.async_copy(
        tdot_hbm.at[idx_v.at[pl.ds(half, half)]],
        val_v.at[pl.ds(half, half)], sem_b)

    cp_a.wait()
    accs = []
    for c in range(_BPW // 16):
        acc = val_v[pl.ds(c * 16, 16)]
        for l in range(1, _L // 2):
            acc = acc + val_v[pl.ds(l * _BPW + c * 16, 16)]
        accs.append(acc)
    cp_b.wait()
    for c in range(_BPW // 16):
        acc = accs[c]
        for l in range(_L // 2, _L):
            acc = acc + val_v[pl.ds(l * _BPW + c * 16, 16)]
        out_v[pl.ds(c * 16, 16)] = 1.0 / (1.0 + jnp.exp(-acc))

    pltpu.sync_copy(out_v, out_hbm.at[pl.ds(base, _BPW)])


def kernel(x, table, W, b):
    table_t = table.astype(jnp.float32).T
    wb = jnp.concatenate(
        [W.astype(jnp.float32),
         b.astype(jnp.float32).reshape(1, 1) * (1.0 / _L)], axis=0)
    tdot = _tdot(table_t, wb)
    idx = (x.astype(jnp.int32)
           .reshape(_NW, _BPW, _L)
           .transpose(0, 2, 1)
           .reshape(_NW, _L * _BPW))
    out = _sc_pool(idx, tdot.reshape(-1))
    return out.reshape(_B, 1)
